# baseline (speedup 1.0000x reference)
import functools

import jax
import jax.numpy as jnp
from jax import lax
from jax.experimental import pallas as pl
from jax.experimental.pallas import tpu as pltpu
from jax.experimental.pallas import tpu_sc as plsc

BATCH = 16384
EMB_DIM = 64
LANES = 16


def kernel(head, relation, tail, ent_emb, rel_emb):
    head = head.reshape(-1).astype(jnp.int32)
    rel = relation.reshape(-1).astype(jnp.int32)
    tail = tail.reshape(-1).astype(jnp.int32)

    info = plsc.get_sparse_core_info()
    nw = info.num_cores * info.num_subcores
    b_per_w = BATCH // nw

    mesh = plsc.VectorSubcoreMesh(core_axis_name="c", subcore_axis_name="s")

    @functools.partial(
        pl.kernel,
        mesh=mesh,
        out_type=jax.ShapeDtypeStruct((BATCH, EMB_DIM), jnp.float32),
        compiler_params=pltpu.CompilerParams(use_tc_tiling_on_sc=False),
        scratch_types=[
            pltpu.VMEM((b_per_w,), jnp.int32),
            pltpu.VMEM((b_per_w,), jnp.int32),
            pltpu.VMEM((b_per_w,), jnp.int32),
            pltpu.VMEM((b_per_w, EMB_DIM), jnp.float32),
            pltpu.VMEM((b_per_w, EMB_DIM), jnp.float32),
            pltpu.VMEM((b_per_w, EMB_DIM), jnp.float32),
            pltpu.SemaphoreType.DMA,
            pltpu.SemaphoreType.DMA,
            pltpu.SemaphoreType.DMA,
        ],
    )
    def trans_e(head_hbm, rel_hbm, tail_hbm, ent_hbm, relemb_hbm, out_hbm,
                hidx, ridx, tidx, hbuf, rbuf, tbuf, sem_h, sem_r, sem_t):
        wid = lax.axis_index("s") * info.num_cores + lax.axis_index("c")
        base = wid * b_per_w

        pltpu.sync_copy(head_hbm.at[pl.ds(base, b_per_w)], hidx)
        pltpu.sync_copy(rel_hbm.at[pl.ds(base, b_per_w)], ridx)
        pltpu.sync_copy(tail_hbm.at[pl.ds(base, b_per_w)], tidx)

        ch = pltpu.async_copy(ent_hbm.at[hidx], hbuf, sem_h)
        cr = pltpu.async_copy(relemb_hbm.at[ridx], rbuf, sem_r)
        ct = pltpu.async_copy(ent_hbm.at[tidx], tbuf, sem_t)
        ch.wait()
        cr.wait()
        ct.wait()

        def row(i, carry):
            for k in range(EMB_DIM // LANES):
                s = pl.ds(k * LANES, LANES)
                hbuf[i, s] = hbuf[i, s] + rbuf[i, s] - tbuf[i, s]
            return carry

        lax.fori_loop(0, b_per_w, row, 0)

        pltpu.sync_copy(hbuf, out_hbm.at[pl.ds(base, b_per_w)])

    return trans_e(head, rel, tail, ent_emb, rel_emb)


# speedup vs baseline: 1.5644x; 1.5644x over previous
import functools

import jax
import jax.numpy as jnp
from jax import lax
from jax.experimental import pallas as pl
from jax.experimental.pallas import tpu as pltpu
from jax.experimental.pallas import tpu_sc as plsc

BATCH = 16384
EMB_DIM = 64
LANES = 16


def kernel(head, relation, tail, ent_emb, rel_emb):
    head = head.reshape(-1).astype(jnp.int32)
    rel = relation.reshape(-1).astype(jnp.int32)
    tail = tail.reshape(-1).astype(jnp.int32)
    n_ent = ent_emb.shape[0]
    n_rel = rel_emb.shape[0]
    ent3 = ent_emb.reshape(n_ent // 8, 8, EMB_DIM)
    rel3 = rel_emb.reshape(n_rel // 8, 8, EMB_DIM)

    info = plsc.get_sparse_core_info()
    nw = info.num_cores * info.num_subcores
    b_per_w = BATCH // nw
    n_groups = b_per_w // LANES

    mesh = plsc.VectorSubcoreMesh(core_axis_name="c", subcore_axis_name="s")

    @functools.partial(
        pl.kernel,
        mesh=mesh,
        out_type=jax.ShapeDtypeStruct((BATCH, EMB_DIM), jnp.float32),
        compiler_params=pltpu.CompilerParams(needs_layout_passes=False),
        scratch_types=[
            pltpu.VMEM((b_per_w,), jnp.int32),
            pltpu.VMEM((b_per_w,), jnp.int32),
            pltpu.VMEM((b_per_w,), jnp.int32),
            pltpu.VMEM((LANES, 8, EMB_DIM), jnp.float32),
            pltpu.VMEM((LANES, 8, EMB_DIM), jnp.float32),
            pltpu.VMEM((LANES, 8, EMB_DIM), jnp.float32),
            pltpu.VMEM((b_per_w, EMB_DIM), jnp.float32),
            pltpu.SemaphoreType.DMA,
            pltpu.SemaphoreType.DMA,
            pltpu.SemaphoreType.DMA,
        ],
    )
    def trans_e(head_hbm, rel_hbm, tail_hbm, ent_hbm, relemb_hbm, out_hbm,
                hidx, ridx, tidx, hbuf, rbuf, tbuf, obuf,
                sem_h, sem_r, sem_t):
        wid = lax.axis_index("s") * info.num_cores + lax.axis_index("c")
        base = wid * b_per_w

        pltpu.sync_copy(head_hbm.at[pl.ds(base, b_per_w)], hidx)
        pltpu.sync_copy(rel_hbm.at[pl.ds(base, b_per_w)], ridx)
        pltpu.sync_copy(tail_hbm.at[pl.ds(base, b_per_w)], tidx)

        lane = lax.iota(jnp.int32, LANES)
        seven = jnp.full((LANES,), 7, jnp.int32)
        zero = jnp.full((LANES,), 0, jnp.int32)

        def group_body(g, carry):
            gs = pl.ds(g * LANES, LANES)
            hch = hidx[gs]
            rch = ridx[gs]
            tch = tidx[gs]
            waits = []
            for l in range(LANES):
                m = lane == l
                hs = lax.reduce_sum(jnp.where(m, hch, zero), axes=(0,))
                rs = lax.reduce_sum(jnp.where(m, rch, zero), axes=(0,))
                ts = lax.reduce_sum(jnp.where(m, tch, zero), axes=(0,))
                waits.append(pltpu.async_copy(
                    ent_hbm.at[lax.shift_right_logical(hs, 3)],
                    hbuf.at[l], sem_h))
                waits.append(pltpu.async_copy(
                    relemb_hbm.at[lax.shift_right_logical(rs, 3)],
                    rbuf.at[l], sem_r))
                waits.append(pltpu.async_copy(
                    ent_hbm.at[lax.shift_right_logical(ts, 3)],
                    tbuf.at[l], sem_t))
            for w in waits:
                w.wait()

            hrow = lax.bitwise_and(hch, seven)
            rrow = lax.bitwise_and(rch, seven)
            trow = lax.bitwise_and(tch, seven)
            oelem = lane + g * LANES

            def dim_body(dd, carry3):
                dvec = zero + dd
                hv = plsc.load_gather(hbuf, [lane, hrow, dvec])
                rv = plsc.load_gather(rbuf, [lane, rrow, dvec])
                tv = plsc.load_gather(tbuf, [lane, trow, dvec])
                plsc.store_scatter(obuf, [oelem, dvec], hv + rv - tv)
                return carry3

            lax.fori_loop(0, EMB_DIM, dim_body, 0)
            return carry

        lax.fori_loop(0, n_groups, group_body, 0)

        pltpu.sync_copy(obuf, out_hbm.at[pl.ds(base, b_per_w)])

    return trans_e(head, rel, tail, ent3, rel3)


# speedup vs baseline: 1.5657x; 1.0008x over previous
import functools

import jax
import jax.numpy as jnp
from jax import lax
from jax.experimental import pallas as pl
from jax.experimental.pallas import tpu as pltpu
from jax.experimental.pallas import tpu_sc as plsc

BATCH = 16384
EMB_DIM = 64
LANES = 16


def kernel(head, relation, tail, ent_emb, rel_emb):
    head = head.reshape(-1).astype(jnp.int32)
    rel = relation.reshape(-1).astype(jnp.int32)
    tail = tail.reshape(-1).astype(jnp.int32)
    n_rel = rel_emb.shape[0]

    info = plsc.get_sparse_core_info()
    nw = info.num_cores * info.num_subcores
    b_per_w = BATCH // nw
    n_groups = b_per_w // LANES

    mesh = plsc.VectorSubcoreMesh(core_axis_name="c", subcore_axis_name="s")

    @functools.partial(
        pl.kernel,
        mesh=mesh,
        out_type=jax.ShapeDtypeStruct((BATCH, EMB_DIM), jnp.float32),
        compiler_params=pltpu.CompilerParams(needs_layout_passes=False),
        scratch_types=[
            pltpu.VMEM((b_per_w,), jnp.int32),
            pltpu.VMEM((b_per_w,), jnp.int32),
            pltpu.VMEM((b_per_w,), jnp.int32),
            pltpu.VMEM((LANES, EMB_DIM), jnp.float32),
            pltpu.VMEM((LANES, EMB_DIM), jnp.float32),
            pltpu.VMEM((LANES, EMB_DIM), jnp.float32),
            pltpu.VMEM((b_per_w, EMB_DIM), jnp.float32),
            pltpu.SemaphoreType.DMA,
            pltpu.SemaphoreType.DMA,
            pltpu.SemaphoreType.DMA,
        ],
    )
    def trans_e(head_hbm, rel_hbm, tail_hbm, ent_hbm, relemb_hbm, out_hbm,
                hidx, ridx, tidx, hbuf, rbuf, tbuf, obuf, sem_h, sem_r, sem_t):
        wid = lax.axis_index("s") * info.num_cores + lax.axis_index("c")
        base = wid * b_per_w

        pltpu.sync_copy(head_hbm.at[pl.ds(base, b_per_w)], hidx)
        pltpu.sync_copy(rel_hbm.at[pl.ds(base, b_per_w)], ridx)
        pltpu.sync_copy(tail_hbm.at[pl.ds(base, b_per_w)], tidx)

        lane = lax.iota(jnp.int32, LANES)
        zero = jnp.full((LANES,), 0, jnp.int32)

        def group_body(g, carry):
            gs = pl.ds(g * LANES, LANES)
            hch = hidx[gs]
            rch = ridx[gs]
            tch = tidx[gs]
            waits = []
            for l in range(LANES):
                m = lane == l
                hs = lax.reduce_sum(jnp.where(m, hch, zero), axes=(0,))
                rs = lax.reduce_sum(jnp.where(m, rch, zero), axes=(0,))
                ts = lax.reduce_sum(jnp.where(m, tch, zero), axes=(0,))
                waits.append(pltpu.async_copy(ent_hbm.at[hs], hbuf.at[l], sem_h))
                waits.append(pltpu.async_copy(relemb_hbm.at[rs], rbuf.at[l], sem_r))
                waits.append(pltpu.async_copy(ent_hbm.at[ts], tbuf.at[l], sem_t))
            for w in waits:
                w.wait()

            for l in range(LANES):
                e = g * LANES + l
                for k in range(EMB_DIM // LANES):
                    s = pl.ds(k * LANES, LANES)
                    obuf[e, s] = hbuf[l, s] + rbuf[l, s] - tbuf[l, s]
            return carry

        lax.fori_loop(0, n_groups, group_body, 0)

        pltpu.sync_copy(obuf, out_hbm.at[pl.ds(base, b_per_w)])

    return trans_e(head, rel, tail, ent_emb, rel_emb)


# speedup vs baseline: 1.5925x; 1.0171x over previous
import functools

import jax
import jax.numpy as jnp
from jax import lax
from jax.experimental import pallas as pl
from jax.experimental.pallas import tpu as pltpu
from jax.experimental.pallas import tpu_sc as plsc

BATCH = 16384
EMB_DIM = 64
LANES = 16


def _scalar(vec, l):
    return lax.squeeze(lax.slice(vec, (l,), (l + 1,)), dimensions=(0,))


def kernel(head, relation, tail, ent_emb, rel_emb):
    head = head.reshape(-1).astype(jnp.int32)
    rel = relation.reshape(-1).astype(jnp.int32)
    tail = tail.reshape(-1).astype(jnp.int32)
    rel_flat = rel_emb.reshape(-1)
    n_rel_words = rel_flat.shape[0]

    info = plsc.get_sparse_core_info()
    nw = info.num_cores * info.num_subcores
    b_per_w = BATCH // nw
    n_groups = b_per_w // LANES

    mesh = plsc.VectorSubcoreMesh(core_axis_name="c", subcore_axis_name="s")

    @functools.partial(
        pl.kernel,
        mesh=mesh,
        out_type=jax.ShapeDtypeStruct((BATCH * EMB_DIM,), jnp.float32),
        compiler_params=pltpu.CompilerParams(needs_layout_passes=False),
        scratch_types=[
            pltpu.VMEM((b_per_w,), jnp.int32),
            pltpu.VMEM((b_per_w,), jnp.int32),
            pltpu.VMEM((b_per_w,), jnp.int32),
            pltpu.VMEM((LANES, EMB_DIM), jnp.float32),
            pltpu.VMEM((LANES, EMB_DIM), jnp.float32),
            pltpu.VMEM((LANES, EMB_DIM), jnp.float32),
            pltpu.VMEM((LANES, EMB_DIM), jnp.float32),
            pltpu.VMEM((n_rel_words,), jnp.float32),
            pltpu.VMEM((b_per_w * EMB_DIM,), jnp.float32),
            pltpu.SemaphoreType.DMA,
            pltpu.SemaphoreType.DMA,
        ],
    )
    def trans_e(head_hbm, rel_hbm, tail_hbm, ent_hbm, relflat_hbm, out_hbm,
                hidx, ridx, tidx, hbuf0, hbuf1, tbuf0, tbuf1, rtab, obuf,
                sem0, sem1):
        wid = lax.axis_index("s") * info.num_cores + lax.axis_index("c")
        base = wid * b_per_w

        pltpu.sync_copy(head_hbm.at[pl.ds(base, b_per_w)], hidx)
        pltpu.sync_copy(rel_hbm.at[pl.ds(base, b_per_w)], ridx)
        pltpu.sync_copy(tail_hbm.at[pl.ds(base, b_per_w)], tidx)
        pltpu.sync_copy(relflat_hbm, rtab)

        def fire(g, hb, tb, sem):
            gs = pl.ds(g * LANES, LANES)
            hch = hidx[gs]
            tch = tidx[gs]
            for l in range(LANES):
                hs = _scalar(hch, l)
                ts = _scalar(tch, l)
                pltpu.async_copy(ent_hbm.at[hs], hb.at[l], sem)
                pltpu.async_copy(ent_hbm.at[ts], tb.at[l], sem)

        def drain(hb, tb, sem):
            for l in range(LANES):
                pltpu.make_async_copy(ent_hbm.at[0], hb.at[l], sem).wait()
                pltpu.make_async_copy(ent_hbm.at[0], tb.at[l], sem).wait()

        def compute(g, hb, tb):
            gs = pl.ds(g * LANES, LANES)
            rch = ridx[gs]
            for l in range(LANES):
                rbase = _scalar(rch, l) * EMB_DIM
                ebase = (g * LANES + l) * EMB_DIM
                for k in range(EMB_DIM // LANES):
                    s = pl.ds(k * LANES, LANES)
                    os_ = pl.ds(ebase + k * LANES, LANES)
                    rs_ = pl.ds(rbase + k * LANES, LANES)
                    obuf[os_] = hb[l, s] + rtab[rs_] - tb[l, s]

        fire(0, hbuf0, tbuf0, sem0)

        def pair_body(p, carry):
            g0 = p * 2
            fire(g0 + 1, hbuf1, tbuf1, sem1)
            drain(hbuf0, tbuf0, sem0)
            compute(g0, hbuf0, tbuf0)

            @pl.when(p < n_groups // 2 - 1)
            def _():
                fire(g0 + 2, hbuf0, tbuf0, sem0)

            drain(hbuf1, tbuf1, sem1)
            compute(g0 + 1, hbuf1, tbuf1)
            return carry

        lax.fori_loop(0, n_groups // 2, pair_body, 0)

        pltpu.sync_copy(obuf, out_hbm.at[pl.ds(base * EMB_DIM, b_per_w * EMB_DIM)])

    out = trans_e(head, rel, tail, ent_emb, rel_flat)
    return out.reshape(BATCH, EMB_DIM)
